# baseline (speedup 1.0000x reference)
import functools

import jax
import jax.numpy as jnp
from jax import lax
from jax.experimental import pallas as pl
from jax.experimental.pallas import tpu as pltpu
from jax.experimental.pallas import tpu_sc as plsc

_N = 5000
_C = 20
_NP = 5120
_CHUNKS = _NP // 16
_NEG = -3.0e38


def _sc_argmax(stp_hbm, tops_hbm, col_vmem, idx_vmem):
    cid = lax.axis_index("c")
    sid = lax.axis_index("s")
    wid = sid * 2 + cid

    @pl.when(wid < _C)
    def _():
        pltpu.sync_copy(stp_hbm.at[wid], col_vmem)
        lanes = lax.broadcasted_iota(jnp.int32, (16,), 0)

        def body(j, carry):
            m, best = carry
            for k in range(8):
                off = j * 128 + k * 16
                v = col_vmem[pl.ds(off, 16)]
                upd = v > m
                m = jnp.where(upd, v, m)
                best = jnp.where(upd, off + lanes, best)
            return m, best

        m0 = jnp.full((16,), _NEG, dtype=jnp.float32)
        b0 = jnp.zeros((16,), dtype=jnp.int32)
        m, best = lax.fori_loop(0, _CHUNKS // 8, body, (m0, b0))

        gmax = jnp.max(m)
        cand = jnp.where(m == gmax, best, _NP)
        idx = jnp.min(cand)
        idx_vmem[...] = jnp.full((16,), idx, dtype=jnp.int32)
        pltpu.sync_copy(idx_vmem, tops_hbm.at[wid])


_sc_call = functools.partial(
    pl.kernel,
    mesh=plsc.VectorSubcoreMesh(core_axis_name="c", subcore_axis_name="s"),
    out_type=jax.ShapeDtypeStruct((_C, 16), jnp.int32),
    scratch_types=[
        pltpu.VMEM((_NP,), jnp.float32),
        pltpu.VMEM((16,), jnp.int32),
    ],
    compiler_params=pltpu.CompilerParams(needs_layout_passes=False),
)(_sc_argmax)


def _tc_body(stp_ref, bt_ref, il_ref, tops_ref, out_ref):
    st = stp_ref[...][:, :_N]
    bt = bt_ref[...]
    idx = tops_ref[:, 0:1]

    col = lax.broadcasted_iota(jnp.int32, st.shape, 1)
    onehot = (col == idx).astype(jnp.float32)
    x1 = bt[0:1, :]
    y1 = bt[1:2, :]
    x2 = bt[2:3, :]
    y2 = bt[3:4, :]
    tx1 = jnp.sum(onehot * x1, axis=1, keepdims=True)
    ty1 = jnp.sum(onehot * y1, axis=1, keepdims=True)
    tx2 = jnp.sum(onehot * x2, axis=1, keepdims=True)
    ty2 = jnp.sum(onehot * y2, axis=1, keepdims=True)

    xx1 = jnp.maximum(x1, tx1)
    yy1 = jnp.maximum(y1, ty1)
    xx2 = jnp.minimum(x2, tx2)
    yy2 = jnp.minimum(y2, ty2)
    iw = xx2 - xx1 + 1.0
    ih = yy2 - yy1 + 1.0
    valid = ((iw > 0) & (ih > 0)).astype(jnp.float32)
    inter = iw * ih * valid
    area_n = (x2 - x1 + 1.0) * (y2 - y1 + 1.0)
    area_t = (tx2 - tx1 + 1.0) * (ty2 - ty1 + 1.0)
    iou = inter / (area_n + area_t - inter)

    cmask = (iou > 0.7).astype(jnp.float32)
    neglog = -jnp.log(jnp.clip(st, 1e-6, 1.0 - 1e-6))
    num = jnp.sum(neglog * cmask, axis=1, keepdims=True)
    den = jnp.maximum(jnp.sum(cmask, axis=1, keepdims=True), 1.0)
    per_class = num / den

    il = il_ref[...]
    loss = jnp.sum(per_class * il) / jnp.sum(il)
    out_ref[...] = jnp.broadcast_to(loss, (1, 1))


@jax.jit
def kernel(boxes, scores, im_labels):
    stp = jnp.pad(scores.T, ((0, 0), (0, _NP - _N)), constant_values=_NEG)
    tops = _sc_call(stp)
    out = pl.pallas_call(
        _tc_body,
        out_shape=jax.ShapeDtypeStruct((1, 1), jnp.float32),
    )(stp, boxes.T, im_labels.T, tops)
    return out[0, 0]


# speedup vs baseline: 1.0040x; 1.0040x over previous
import functools

import jax
import jax.numpy as jnp
from jax import lax
from jax.experimental import pallas as pl
from jax.experimental.pallas import tpu as pltpu
from jax.experimental.pallas import tpu_sc as plsc

_N = 5000
_C = 20
_NP = 5120
_NEG = -3.0e38


def _sc_argmax(stp_hbm, tops_hbm, col_vmem, idx_vmem):
    cid = lax.axis_index("c")
    sid = lax.axis_index("s")
    wid = sid * 2 + cid

    @pl.when(wid < _C)
    def _():
        pltpu.sync_copy(stp_hbm.at[wid], col_vmem)
        lanes = lax.broadcasted_iota(jnp.int32, (16,), 0)

        def step(off, v, carry):
            m, best = carry
            upd = v > m
            return jnp.where(upd, v, m), jnp.where(upd, off + lanes, best)

        def body(j, carry):
            for k in range(8):
                off = j * 128 + k * 16
                carry = step(off, col_vmem[pl.ds(off, 16)], carry)
            return carry

        m0 = jnp.full((16,), _NEG, dtype=jnp.float32)
        b0 = jnp.zeros((16,), dtype=jnp.int32)
        m, best = lax.fori_loop(0, _NP // 128, body, (m0, b0))

        gmax = jnp.max(m)
        cand = jnp.where(m == gmax, best, _N)
        idx = jnp.min(cand)
        idx_vmem[...] = jnp.full((16,), idx, dtype=jnp.int32)
        pltpu.sync_copy(idx_vmem, tops_hbm.at[wid])


_sc_call = functools.partial(
    pl.kernel,
    mesh=plsc.VectorSubcoreMesh(core_axis_name="c", subcore_axis_name="s"),
    out_type=jax.ShapeDtypeStruct((_C, 16), jnp.int32),
    scratch_types=[
        pltpu.VMEM((_NP,), jnp.float32),
        pltpu.VMEM((16,), jnp.int32),
    ],
    compiler_params=pltpu.CompilerParams(needs_layout_passes=False),
)(_sc_argmax)


def _tc_body(stp_ref, bt_ref, il_ref, tops_ref, out_ref):
    st = stp_ref[...][:, :_N]
    bt = bt_ref[...]
    idx = tops_ref[:, 0:1]

    col = lax.broadcasted_iota(jnp.int32, st.shape, 1)
    onehot = (col == idx).astype(jnp.float32)
    x1 = bt[0:1, :]
    y1 = bt[1:2, :]
    x2 = bt[2:3, :]
    y2 = bt[3:4, :]
    tx1 = jnp.sum(onehot * x1, axis=1, keepdims=True)
    ty1 = jnp.sum(onehot * y1, axis=1, keepdims=True)
    tx2 = jnp.sum(onehot * x2, axis=1, keepdims=True)
    ty2 = jnp.sum(onehot * y2, axis=1, keepdims=True)

    xx1 = jnp.maximum(x1, tx1)
    yy1 = jnp.maximum(y1, ty1)
    xx2 = jnp.minimum(x2, tx2)
    yy2 = jnp.minimum(y2, ty2)
    iw = xx2 - xx1 + 1.0
    ih = yy2 - yy1 + 1.0
    valid = ((iw > 0) & (ih > 0)).astype(jnp.float32)
    inter = iw * ih * valid
    area_n = (x2 - x1 + 1.0) * (y2 - y1 + 1.0)
    area_t = (tx2 - tx1 + 1.0) * (ty2 - ty1 + 1.0)
    iou = inter / (area_n + area_t - inter)

    cmask = (iou > 0.7).astype(jnp.float32)
    neglog = -jnp.log(jnp.clip(st, 1e-6, 1.0 - 1e-6))
    num = jnp.sum(neglog * cmask, axis=1, keepdims=True)
    den = jnp.maximum(jnp.sum(cmask, axis=1, keepdims=True), 1.0)
    per_class = num / den

    il = il_ref[...]
    loss = jnp.sum(per_class * il) / jnp.sum(il)
    out_ref[...] = jnp.broadcast_to(loss, (1, 1))


@jax.jit
def kernel(boxes, scores, im_labels):
    stp = jnp.pad(scores.T, ((0, 0), (0, _NP - _N)), constant_values=_NEG)
    tops = _sc_call(stp)
    out = pl.pallas_call(
        _tc_body,
        out_shape=jax.ShapeDtypeStruct((1, 1), jnp.float32),
    )(stp, boxes.T, im_labels.T, tops)
    return out[0, 0]
